# baseline (speedup 1.0000x reference)
import functools

import jax
import jax.numpy as jnp
from jax import lax
from jax.experimental import pallas as pl
from jax.experimental.pallas import tpu as pltpu
from jax.experimental.pallas import tpu_sc as plsc

N = 4096
D = 128
K = 16
H = 2 * (2 * D + 1)
MD = 16
CP = 16

RA = 256
CK = 32
TPC = 6
RC = 256
E = K * RC

NC = 2
NS = 16
NW = NC * NS
B = N * K
BPW = B // NW
CH = 128
NCH = BPW // CH

def _silu(x):
    return x * jax.nn.sigmoid(x)


def _topk_body(cp_ref, cit_ref, f_ref, cr_ref, idx_ref, tab_ref):
    gi = pl.program_id(0)
    cj = cp_ref[...]
    ci = cit_ref[...]
    tab_ref[...] = jnp.concatenate(
        [f_ref[...], cr_ref[...], jnp.zeros((RA, CP - 3), jnp.float32)],
        axis=1)
    d0 = cj[:, 0:1] - ci[0:1, :]
    d1 = cj[:, 1:2] - ci[1:2, :]
    d2 = cj[:, 2:3] - ci[2:3, :]
    dist = d0 * d0 + d1 * d1 + d2 * d2
    jiota = lax.broadcasted_iota(jnp.int32, (N, RA), 0).astype(jnp.float32)
    rowf = (lax.broadcasted_iota(jnp.int32, (1, RA), 1).astype(jnp.float32)
            + jnp.float32(RA) * gi.astype(jnp.float32))
    dist = jnp.where(jiota == rowf, 1e30, dist)
    CS = N // CK
    cd, ci_l = [], []
    for c in range(CK):
        dch = dist[c * CS:(c + 1) * CS, :]
        jch = jiota[c * CS:(c + 1) * CS, :]
        for _ in range(TPC):
            m = jnp.min(dch, axis=0, keepdims=True)
            loc = jnp.min(jnp.where(dch <= m, jch, jnp.float32(N)),
                          axis=0, keepdims=True)
            cd.append(m)
            ci_l.append(loc)
            dch = jnp.where(jch == loc, 1e30, dch)
    cand_d = jnp.concatenate(cd, axis=0)
    cand_i = jnp.concatenate(ci_l, axis=0)
    cols = [rowf]
    for k in range(K - 1):
        m = jnp.min(cand_d, axis=0, keepdims=True)
        idx = jnp.min(jnp.where(cand_d <= m, cand_i, jnp.float32(N)),
                      axis=0, keepdims=True)
        cols.append(idx)
        if k < K - 2:
            cand_d = jnp.where(cand_i == idx, 1e30, cand_d)
    idx_ref[...] = jnp.concatenate(cols, axis=0).astype(jnp.int32)


def _topk(coors, coors_t, f):
    return pl.pallas_call(
        _topk_body,
        grid=(N // RA,),
        in_specs=[
            pl.BlockSpec((N, 3), lambda i: (0, 0)),
            pl.BlockSpec((8, RA), lambda i: (0, i)),
            pl.BlockSpec((RA, D), lambda i: (i, 0)),
            pl.BlockSpec((RA, 3), lambda i: (i, 0)),
        ],
        out_specs=[
            pl.BlockSpec((K, RA), lambda i: (0, i)),
            pl.BlockSpec((RA, D + CP), lambda i: (i, 0)),
        ],
        out_shape=[
            jax.ShapeDtypeStruct((K, N), jnp.int32),
            jax.ShapeDtypeStruct((N, D + CP), jnp.float32),
        ],
    )(coors, coors_t, f, coors)


@functools.cache
def _make_sc_gather():
    mesh = plsc.VectorSubcoreMesh(core_axis_name="c", subcore_axis_name="s")

    @functools.partial(
        pl.kernel,
        mesh=mesh,
        out_type=[
            jax.ShapeDtypeStruct((B, D), jnp.float32),
            jax.ShapeDtypeStruct((B, CP), jnp.float32),
        ],
        scratch_types=[
            pltpu.VMEM((NCH, CH), jnp.int32),
            pltpu.VMEM((CH, D + CP), jnp.float32),
            pltpu.VMEM((CH, D + CP), jnp.float32),
            pltpu.SemaphoreType.DMA,
            pltpu.SemaphoreType.DMA,
            pltpu.SemaphoreType.DMA,
            pltpu.SemaphoreType.DMA,
        ],
        compiler_params=pltpu.CompilerParams(use_tc_tiling_on_sc=False),
    )
    def _sc_gather(table_hbm, idx_hbm, outf_hbm, outc_hbm,
                   idx_v, rows0, rows1, gs0, gs1, ws0, ws1):
        wid = lax.axis_index("s") * NC + lax.axis_index("c")
        base = wid * NCH
        pltpu.sync_copy(idx_hbm.at[pl.ds(base, NCH)], idx_v)

        bufs = (rows0, rows1)
        gsems = (gs0, gs1)
        wsems = (ws0, ws1)
        gh = [None] * NCH
        wh = [None] * NCH
        for c in range(NCH):
            b = c & 1
            if c >= 2:
                for h in wh[c - 2]:
                    h.wait()
            gh[c] = pltpu.async_copy(table_hbm.at[idx_v.at[c]], bufs[b],
                                     gsems[b])
            if c >= 1:
                pb = (c - 1) & 1
                gh[c - 1].wait()
                dst = pl.ds((base + c - 1) * CH, CH)
                wh[c - 1] = (
                    pltpu.async_copy(bufs[pb].at[:, pl.ds(0, D)],
                                     outf_hbm.at[dst], wsems[pb]),
                    pltpu.async_copy(bufs[pb].at[:, pl.ds(D, CP)],
                                     outc_hbm.at[dst], wsems[pb]),
                )
        c = NCH - 1
        gh[c].wait()
        dst = pl.ds((base + c) * CH, CH)
        wh[c] = (
            pltpu.async_copy(bufs[c & 1].at[:, pl.ds(0, D)],
                             outf_hbm.at[dst], wsems[c & 1]),
            pltpu.async_copy(bufs[c & 1].at[:, pl.ds(D, CP)],
                             outc_hbm.at[dst], wsems[c & 1]),
        )
        for h in wh[NCH - 2]:
            h.wait()
        for h in wh[NCH - 1]:
            h.wait()

    return _sc_gather


def _mlp_body(gf_ref, gc_ref, f_ref, cp_ref, w1a_ref, w1b_ref, w1d_ref, b1_ref,
              w2_ref, b2_ref, wc1_ref, bc1_ref, wc2_ref, bc2_ref,
              wn1a_ref, wn1b_ref, bn1_ref, wn2_ref, bn2_ref,
              node_ref, coor_ref):
    f = f_ref[...]
    craw = cp_ref[...]
    cp = jnp.concatenate(
        [craw, jnp.zeros((RC, CP - 3), jnp.float32)], axis=1)
    gc = gc_ref[...]
    fj = gf_ref[...].reshape(E, D)
    ai = jnp.dot(f, w1a_ref[...]) + b1_ref[...]
    aib = jnp.broadcast_to(ai[None], (K, RC, H)).reshape(E, H)
    cpb = jnp.broadcast_to(cp[None], (K, RC, CP))
    rel3 = cpb - gc
    rd = jnp.sum(rel3 * rel3, axis=2, keepdims=True).reshape(E, 1)
    pre = jnp.dot(fj, w1b_ref[...])
    h1 = _silu(aib + pre + rd * w1d_ref[...])
    m = _silu(jnp.dot(h1, w2_ref[...]) + b2_ref[...])
    c1 = _silu(jnp.dot(m, wc1_ref[...]) + bc1_ref[...])
    cw = jnp.dot(c1, wc2_ref[...]) + bc2_ref[...]
    mi = jnp.sum(m.reshape(K, RC, MD), axis=0)
    csum = jnp.sum(cw.reshape(K, RC, 1) * rel3, axis=0)
    coor_ref[...] = csum[:, 0:3] + craw
    t = _silu(jnp.dot(f, wn1a_ref[...])
              + jnp.dot(mi, wn1b_ref[...]) + bn1_ref[...])
    node_ref[...] = jnp.dot(t, wn2_ref[...]) + bn2_ref[...] + f


def _mlp(gf3, gc3, f, cp, w1a, w1b, w1d, b1, w2, b2, wc1, bc1, wc2, bc2,
         wn1a, wn1b, bn1, wn2, bn2):
    full = lambda shape: pl.BlockSpec(shape, lambda i: tuple(0 for _ in shape))
    blk = lambda shape: pl.BlockSpec(shape, lambda i: (i, 0))
    blk3 = lambda shape: pl.BlockSpec(shape, lambda i: (0, i, 0))
    return pl.pallas_call(
        _mlp_body,
        grid=(N // RC,),
        in_specs=[
            blk3((K, RC, D)),
            blk3((K, RC, CP)),
            blk((RC, D)),
            blk((RC, 3)),
            full((D, H)), full((D, H)), full((1, H)), full((1, H)),
            full((H, MD)), full((1, MD)),
            full((MD, 4 * MD)), full((1, 4 * MD)),
            full((4 * MD, 1)), full((1, 1)),
            full((D, 2 * D)), full((MD, 2 * D)), full((1, 2 * D)),
            full((2 * D, D)), full((1, D)),
        ],
        out_specs=[blk((RC, D)), blk((RC, 3))],
        out_shape=[
            jax.ShapeDtypeStruct((N, D), jnp.float32),
            jax.ShapeDtypeStruct((N, 3), jnp.float32),
        ],
    )(gf3, gc3, f, cp, w1a, w1b, w1d, b1, w2, b2, wc1, bc1, wc2, bc2,
      wn1a, wn1b, bn1, wn2, bn2)


def kernel(feats, coors, W1, b1, W2, b2, Wc1, bc1, Wc2, bc2, Wn1, bn1, Wn2, bn2):
    f = feats[0]
    c = coors[0]
    ct = jnp.pad(c.T, ((0, 5), (0, 0)))

    idx, table = _topk(c, ct, f)
    idx2d = idx.reshape(B // CH, CH)

    gf, gc = _make_sc_gather()(table, idx2d)
    gf3 = gf.reshape(K, N, D)
    gc3 = gc.reshape(K, N, CP)

    node, coor3 = _mlp(
        gf3, gc3, f, c,
        W1[:D], W1[D:2 * D], W1[2 * D:2 * D + 1], b1[None],
        W2, b2[None],
        Wc1, bc1[None],
        Wc2, bc2[None],
        Wn1[:D], Wn1[D:], bn1[None],
        Wn2, bn2[None],
    )
    return node[None], coor3[None]
